# baseline (speedup 1.0000x reference)
import functools

import jax
import jax.numpy as jnp
from jax import lax
from jax.experimental import pallas as pl
from jax.experimental.pallas import tpu as pltpu
from jax.experimental.pallas import tpu_sc as plsc

VOCAB = 1000000
D = 64
B = 1024
L = 200
EPS = 1e-5

NC = 2
NS = 16
NW = NC * NS
N = B * L
RPW = N // NW
CH = 128
NCH = RPW // CH


def _ln_rows(rows_v, b, pe_v, g, bt, r0):
    g0, g1, g2, g3 = g
    b0, b1, b2, b3 = bt

    def row(i, p):
        x0 = rows_v[b, i, pl.ds(0, 16)] + pe_v[p, pl.ds(0, 16)]
        x1 = rows_v[b, i, pl.ds(16, 16)] + pe_v[p, pl.ds(16, 16)]
        x2 = rows_v[b, i, pl.ds(32, 16)] + pe_v[p, pl.ds(32, 16)]
        x3 = rows_v[b, i, pl.ds(48, 16)] + pe_v[p, pl.ds(48, 16)]
        s = (x0 + x1) + (x2 + x3)
        q = (x0 * x0 + x1 * x1) + (x2 * x2 + x3 * x3)
        mean = jnp.sum(s) * (1.0 / D)
        ex2 = jnp.sum(q) * (1.0 / D)
        t = ex2 - mean * mean + EPS
        ti = lax.bitcast_convert_type(t, jnp.int32)
        y = lax.bitcast_convert_type(jnp.int32(0x5F3759DF) - (ti >> 1),
                                     jnp.float32)
        y = y * (1.5 - 0.5 * t * y * y)
        y = y * (1.5 - 0.5 * t * y * y)
        y = y * (1.5 - 0.5 * t * y * y)
        rows_v[b, i, pl.ds(0, 16)] = (x0 - mean) * y * g0 + b0
        rows_v[b, i, pl.ds(16, 16)] = (x1 - mean) * y * g1 + b1
        rows_v[b, i, pl.ds(32, 16)] = (x2 - mean) * y * g2 + b2
        rows_v[b, i, pl.ds(48, 16)] = (x3 - mean) * y * g3 + b3
        pn = p + 1
        return lax.select(pn == L, 0, pn)

    lax.fori_loop(0, CH, row, r0)


def _embed_ln(tok_hbm, table_hbm, pe_hbm, gb_hbm, out_hbm,
              idx_v, rows_v, pe_v, gb_v,
              psem, gsem0, gsem1, ssem0, ssem1):
    wid = lax.axis_index("s") * NC + lax.axis_index("c")
    base = pl.multiple_of(wid * RPW, 8)

    pltpu.async_copy(pe_hbm, pe_v, psem)
    pltpu.async_copy(gb_hbm, gb_v, psem).wait()
    pltpu.make_async_copy(pe_hbm, pe_v, psem).wait()

    g = tuple(gb_v[pl.ds(16 * j, 16)] for j in range(4))
    bt = tuple(gb_v[pl.ds(D + 16 * j, 16)] for j in range(4))

    gsems = (gsem0, gsem1)
    ssems = (ssem0, ssem1)

    def start_gather(k, b, sem):
        off = pl.multiple_of(base + k * CH, 8)
        pltpu.sync_copy(tok_hbm.at[pl.ds(off, CH)], idx_v.at[b])
        pltpu.async_copy(table_hbm.at[idx_v.at[b]], rows_v.at[b], sem)

    def wait_gather(b, sem):
        pltpu.make_async_copy(table_hbm.at[pl.ds(0, CH)], rows_v.at[b],
                              sem).wait()

    def start_scatter(k, b, sem):
        off = pl.multiple_of(base + k * CH, 8)
        pltpu.async_copy(rows_v.at[b], out_hbm.at[pl.ds(off, CH)], sem)

    def wait_scatter(b, sem):
        pltpu.make_async_copy(rows_v.at[b], out_hbm.at[pl.ds(0, CH)],
                              sem).wait()

    start_gather(0, 0, gsems[0])

    def step(k, b):
        nb = 1 - b

        @pl.when(k + 1 < NCH)
        def _prefetch():
            @pl.when(k >= 1)
            def _drain():
                wait_scatter(nb, ssems[nb])
            start_gather(k + 1, nb, gsems[nb])

        wait_gather(b, gsems[b])
        r0 = lax.rem(k * CH, L)
        _ln_rows(rows_v, b, pe_v, g, bt, r0)
        start_scatter(k, b, ssems[b])

    def pair(p, _):
        step(2 * p, 0)
        step(2 * p + 1, 1)
        return 0

    lax.fori_loop(0, NCH // 2, pair, 0)
    wait_scatter(0, ssems[0])
    wait_scatter(1, ssems[1])


@jax.jit
def _run(tok_flat, table, pe_rows, gb):
    mesh = plsc.VectorSubcoreMesh(core_axis_name="c", subcore_axis_name="s")
    return pl.kernel(
        _embed_ln,
        out_type=jax.ShapeDtypeStruct((N, D), jnp.float32),
        mesh=mesh,
        scratch_types=[
            pltpu.VMEM((2, CH), jnp.int32),
            pltpu.VMEM((2, CH, D), jnp.float32),
            pltpu.VMEM((L, D), jnp.float32),
            pltpu.VMEM((2 * D,), jnp.float32),
            pltpu.SemaphoreType.DMA,
            pltpu.SemaphoreType.DMA,
            pltpu.SemaphoreType.DMA,
            pltpu.SemaphoreType.DMA,
            pltpu.SemaphoreType.DMA,
        ],
        compiler_params=pltpu.CompilerParams(needs_layout_passes=False,
                                             use_tc_tiling_on_sc=False),
    )(tok_flat, table, pe_rows, gb)


def kernel(token_ids, table, gamma, beta, pe):
    tok_flat = token_ids.astype(jnp.int32).reshape(-1)
    pe_rows = pe[0, :L, :].astype(jnp.float32)
    gb = jnp.concatenate([gamma, beta]).astype(jnp.float32)
    out = _run(tok_flat, table, pe_rows, gb)
    return out.reshape(B, L, D)


# speedup vs baseline: 1.3162x; 1.3162x over previous
import functools

import jax
import jax.numpy as jnp
from jax import lax
from jax.experimental import pallas as pl
from jax.experimental.pallas import tpu as pltpu
from jax.experimental.pallas import tpu_sc as plsc

VOCAB = 1000000
D = 64
B = 1024
L = 200
EPS = 1e-5

NC = 2
NS = 16
NW = NC * NS
N = B * L
RPW = N // NW
CH = 256
NCH = RPW // CH
NSTR = CH // 128


def _ln_rows(rows_v, b, pe_v, g, bt, r0):
    g0, g1, g2, g3 = g
    b0, b1, b2, b3 = bt

    @plsc.parallel_loop(0, CH, step=1, unroll=8)
    def _row(i):
        p = lax.rem(r0 + i, L)
        x0 = rows_v[b, i, pl.ds(0, 16)] + pe_v[p, pl.ds(0, 16)]
        x1 = rows_v[b, i, pl.ds(16, 16)] + pe_v[p, pl.ds(16, 16)]
        x2 = rows_v[b, i, pl.ds(32, 16)] + pe_v[p, pl.ds(32, 16)]
        x3 = rows_v[b, i, pl.ds(48, 16)] + pe_v[p, pl.ds(48, 16)]
        s = (x0 + x1) + (x2 + x3)
        q = (x0 * x0 + x1 * x1) + (x2 * x2 + x3 * x3)
        mean = jnp.sum(s) * (1.0 / D)
        ex2 = jnp.sum(q) * (1.0 / D)
        t = ex2 - mean * mean + EPS
        ti = lax.bitcast_convert_type(t, jnp.int32)
        y = lax.bitcast_convert_type(jnp.int32(0x5F3759DF) - (ti >> 1),
                                     jnp.float32)
        y = y * (1.5 - 0.5 * t * y * y)
        y = y * (1.5 - 0.5 * t * y * y)
        rows_v[b, i, pl.ds(0, 16)] = (x0 - mean) * (y * g0) + b0
        rows_v[b, i, pl.ds(16, 16)] = (x1 - mean) * (y * g1) + b1
        rows_v[b, i, pl.ds(32, 16)] = (x2 - mean) * (y * g2) + b2
        rows_v[b, i, pl.ds(48, 16)] = (x3 - mean) * (y * g3) + b3


def _embed_ln(tok_hbm, table_hbm, pe_hbm, gb_hbm, out_hbm,
              idx_v, rows_v, pe_v, gb_v,
              psem, gsem0, gsem1, ssem0, ssem1):
    wid = lax.axis_index("s") * NC + lax.axis_index("c")
    base = pl.multiple_of(wid * RPW, 256)

    pltpu.async_copy(pe_hbm, pe_v, psem)
    pltpu.async_copy(gb_hbm, gb_v, psem)
    pltpu.async_copy(tok_hbm.at[pl.ds(base, RPW)], idx_v, psem).wait()
    pltpu.make_async_copy(pe_hbm, pe_v, psem).wait()
    pltpu.make_async_copy(gb_hbm, gb_v, psem).wait()

    g = tuple(gb_v[pl.ds(16 * j, 16)] for j in range(4))
    bt = tuple(gb_v[pl.ds(D + 16 * j, 16)] for j in range(4))

    gsems = (gsem0, gsem1)
    ssems = (ssem0, ssem1)

    def start_gather(k, b, sem):
        for j in range(NSTR):
            ioff = pl.multiple_of(k * CH + j * 128, 128)
            pltpu.async_copy(table_hbm.at[idx_v.at[pl.ds(ioff, 128)]],
                             rows_v.at[b, pl.ds(j * 128, 128)], sem)

    def wait_gather(b, sem):
        pltpu.make_async_copy(table_hbm.at[pl.ds(0, CH)], rows_v.at[b],
                              sem).wait()

    def start_scatter(k, b, sem):
        off = pl.multiple_of(base + k * CH, 256)
        pltpu.async_copy(rows_v.at[b], out_hbm.at[pl.ds(off, CH)], sem)

    def wait_scatter(b, sem):
        pltpu.make_async_copy(rows_v.at[b], out_hbm.at[pl.ds(0, CH)],
                              sem).wait()

    start_gather(0, 0, gsems[0])

    def step(k, b, prefetch):
        nb = 1 - b

        if prefetch:
            @pl.when(k >= 1)
            def _drain():
                wait_scatter(nb, ssems[nb])
            start_gather(k + 1, nb, gsems[nb])

        wait_gather(b, gsems[b])
        r0 = lax.rem(k * CH, L)
        _ln_rows(rows_v, b, pe_v, g, bt, r0)
        start_scatter(k, b, ssems[b])

    def pair(p, _):
        step(2 * p, 0, True)
        step(2 * p + 1, 1, True)
        return 0

    lax.fori_loop(0, NCH // 2, pair, 0)
    step(NCH - 1, 0, False)
    wait_scatter(1, ssems[1])
    wait_scatter(0, ssems[0])


@jax.jit
def _run(tok_flat, table, pe_rows, gb):
    mesh = plsc.VectorSubcoreMesh(core_axis_name="c", subcore_axis_name="s")
    return pl.kernel(
        _embed_ln,
        out_type=jax.ShapeDtypeStruct((N, D), jnp.float32),
        mesh=mesh,
        scratch_types=[
            pltpu.VMEM((RPW,), jnp.int32),
            pltpu.VMEM((2, CH, D), jnp.float32),
            pltpu.VMEM((L, D), jnp.float32),
            pltpu.VMEM((2 * D,), jnp.float32),
            pltpu.SemaphoreType.DMA,
            pltpu.SemaphoreType.DMA,
            pltpu.SemaphoreType.DMA,
            pltpu.SemaphoreType.DMA,
            pltpu.SemaphoreType.DMA,
        ],
        compiler_params=pltpu.CompilerParams(needs_layout_passes=False,
                                             use_tc_tiling_on_sc=False),
    )(tok_flat, table, pe_rows, gb)


def kernel(token_ids, table, gamma, beta, pe):
    tok_flat = token_ids.astype(jnp.int32).reshape(-1)
    pe_rows = pe[0, :L, :].astype(jnp.float32)
    gb = jnp.concatenate([gamma, beta]).astype(jnp.float32)
    out = _run(tok_flat, table, pe_rows, gb)
    return out.reshape(B, L, D)
